# baseline (speedup 1.0000x reference)
import functools

import jax
import jax.numpy as jnp
from jax import lax
from jax.experimental import pallas as pl
from jax.experimental.pallas import tpu as pltpu
from jax.experimental.pallas import tpu_sc as plsc

_N = 10000
_E = 160000
_D = 128
_EDIM = 16
_L = 4
_AVG_D_LOG = 2.772588722239781

_NT = 32
_NPT = 360
_NPAD = _NT * _NPT
_NSB = 5
_SUB = 72
_EB = 64
_EPAD = 162000
_EPB = 200

_NBLK = 1440
_EBLK = 2000


def _make_sc_permute():
  mesh = plsc.VectorSubcoreMesh(
      core_axis_name="c", subcore_axis_name="s", num_cores=2,
      num_subcores=16)
  ept = _E // _NT

  def body(e, pm, es, pmb, erb, sem):
    wid = lax.axis_index("s") * 2 + lax.axis_index("c")
    r0 = wid * ept

    def blk(k, carry):
      bs = r0 + k * _EPB
      pltpu.sync_copy(pm.at[pl.ds(bs, _EPB)], pmb)
      pltpu.async_copy(e.at[pmb], erb, sem).wait()
      pltpu.sync_copy(erb, es.at[pl.ds(bs, _EPB)])
      return carry

    lax.fori_loop(0, ept // _EPB, blk, 0)

  return pl.kernel(
      body,
      out_type=jax.ShapeDtypeStruct((_EPAD, _D), jnp.float32),
      mesh=mesh,
      scratch_types=[
          pltpu.VMEM((_EPB,), jnp.int32),
          pltpu.VMEM((_EPB, _D), jnp.float32),
          pltpu.SemaphoreType.DMA,
      ])


def _make_sc_edge(want_cnt):
  mesh = plsc.VectorSubcoreMesh(
      core_axis_name="c", subcore_axis_name="s", num_cores=2,
      num_subcores=16)
  out_type = [jax.ShapeDtypeStruct((_NPAD, _D), jnp.float32) for _ in range(4)]
  if want_cnt:
    out_type.append(jax.ShapeDtypeStruct((_NPAD, 16), jnp.float32))
  scratch = [
      [pltpu.VMEM((_EB,), jnp.int32)] * 4,
      [pltpu.VMEM((_EB + 16,), jnp.int32)] * 4,
      [pltpu.VMEM((_EB, _D), jnp.float32)] * 4,
      [pltpu.VMEM((_EB, _D), jnp.float32)] * 4,
      pltpu.VMEM((_SUB, _D), jnp.float32),
      pltpu.VMEM((_SUB, _D), jnp.float32),
      pltpu.VMEM((_SUB, _D), jnp.float32),
      pltpu.VMEM((_SUB, _D), jnp.float32),
      pltpu.VMEM((_SUB, _D), jnp.float32),
      pltpu.VMEM((_SUB, 16), jnp.float32),
      pltpu.VMEM((40, 16), jnp.float32),
      pltpu.VMEM((1, 32), jnp.int32),
      [pltpu.SemaphoreType.DMA] * 4,
      [pltpu.SemaphoreType.DMA] * 4,
      [pltpu.SemaphoreType.DMA] * 4,
      pltpu.SemaphoreType.DMA,
  ]

  def body(g1, g2, q, ss, fn, tr, *refs):
    nout = 5 if want_cnt else 4
    outs = refs[:nout]
    (ssb, fnb, g1b, qb, g2s, sumb, sqb, mxb, mnb, cntb, accbuf, rngb,
     sem_gidx, sem_fnb, sem_dat, sem_g2s) = refs[nout:]
    if want_cnt:
      s_out, q_out, mx_out, mn_out, cnt_out = outs
    else:
      s_out, q_out, mx_out, mn_out = outs

    wid = lax.axis_index("s") * 2 + lax.axis_index("c")
    pltpu.sync_copy(tr.at[wid], rngb.at[0])

    zeros16 = jnp.zeros((16,), jnp.float32)
    big16 = jnp.full((16,), 1e30, jnp.float32)

    def issue_gidx(b, bs):
      pltpu.async_copy(ss.at[pl.ds(bs, _EB)], ssb[b], sem_gidx[b])

    def wait_gidx(b):
      pltpu.make_async_copy(ss.at[pl.ds(0, _EB)], ssb[b], sem_gidx[b]).wait()

    def issue_fnb(b, bs):
      pltpu.async_copy(fn.at[pl.ds(bs, _EB)], fnb[b].at[pl.ds(0, _EB)],
                       sem_fnb[b])

    def wait_fnb(b):
      pltpu.make_async_copy(fn.at[pl.ds(0, _EB)], fnb[b].at[pl.ds(0, _EB)],
                            sem_fnb[b]).wait()

    def issue_dat(b, bs):
      pltpu.async_copy(g1.at[ssb[b]], g1b[b], sem_dat[b])
      pltpu.async_copy(q.at[pl.ds(bs, _EB)], qb[b], sem_dat[b])

    def wait_dat(b):
      pltpu.make_async_copy(g1.at[ssb[b]], g1b[b], sem_dat[b]).wait()
      pltpu.make_async_copy(q.at[pl.ds(0, _EB)], qb[b], sem_dat[b]).wait()

    def sub_body(j, _):
      nlo = wid * _NPT + j * _SUB

      cpg = pltpu.async_copy(g2.at[pl.ds(nlo, _SUB)], g2s, sem_g2s)

      def init_body(r, carry):
        for c in range(8):
          sl = pl.ds(c * 16, 16)
          sumb[r, sl] = zeros16
          sqb[r, sl] = zeros16
          mxb[r, sl] = zeros16
          mnb[r, sl] = big16
        if want_cnt:
          cntb[r, pl.ds(0, 16)] = zeros16
        return carry

      lax.fori_loop(0, _SUB, init_body, 0)

      rv = rngb[0, pl.ds(j, 16)]
      e0 = rv[0]
      e1 = rv[1]
      base = (e0 // _EB) * _EB
      nblk = (e1 - base + (_EB - 1)) // _EB
      nquad = jnp.maximum((nblk + 3) // 4, 1)

      for b in range(4):
        issue_gidx(b, base + b * _EB)
      for b in range(3):
        wait_gidx(b)
        issue_dat(b, base + b * _EB)
        issue_fnb(b, base + b * _EB)
      cpg.wait()

      sl16 = pl.ds(0, 16)

      def store_accs(accs):
        sacc, qacc, xacc, nacc, cacc = accs
        for c in range(8):
          accbuf[c, sl16] = sacc[c]
          accbuf[8 + c, sl16] = qacc[c]
          accbuf[16 + c, sl16] = xacc[c]
          accbuf[24 + c, sl16] = nacc[c]
        accbuf[32, sl16] = cacc

      def load_accs():
        return ([accbuf[c, sl16] for c in range(8)],
                [accbuf[8 + c, sl16] for c in range(8)],
                [accbuf[16 + c, sl16] for c in range(8)],
                [accbuf[24 + c, sl16] for c in range(8)],
                accbuf[32, sl16])

      store_accs(([zeros16] * 8, [zeros16] * 8, [zeros16] * 8, [big16] * 8,
                  zeros16))

      def edge_step(b, el, v, accs):
        sacc, qacc, xacc, nacc, cacc = accs
        row = (v >> 1) - nlo
        is_b = (v & 1) == 1
        sacc2, qacc2, xacc2, nacc2 = [], [], [], []
        for c in range(8):
          sl = pl.ds(c * 16, 16)
          m = g1b[b][el, sl] + g2s[row, sl] + qb[b][el, sl]
          m = jnp.maximum(m, 0.0)
          sacc2.append(sacc[c] + m)
          qacc2.append(qacc[c] + m * m)
          xacc2.append(jnp.maximum(xacc[c], m))
          nacc2.append(jnp.minimum(nacc[c], m))
        cacc2 = cacc + jnp.maximum(m * 0.0, 1.0)

        @pl.when(is_b)
        def _():
          for c in range(8):
            sl = pl.ds(c * 16, 16)
            sumb[row, sl] = sacc2[c]
            sqb[row, sl] = qacc2[c]
            mxb[row, sl] = xacc2[c]
            mnb[row, sl] = nacc2[c]
          if want_cnt:
            cntb[row, pl.ds(0, 16)] = cacc2

        sacc3 = [jnp.where(is_b, zeros16, x) for x in sacc2]
        qacc3 = [jnp.where(is_b, zeros16, x) for x in qacc2]
        xacc3 = [jnp.where(is_b, zeros16, x) for x in xacc2]
        nacc3 = [jnp.where(is_b, big16, x) for x in nacc2]
        cacc3 = jnp.where(is_b, zeros16, cacc2)
        return (sacc3, qacc3, xacc3, nacc3, cacc3)

      def process(kb, b):
        bs = base + kb * _EB
        lo = jnp.maximum(e0 - bs, 0)
        hi = jnp.minimum(e1 - bs, _EB)
        full = jnp.logical_and(lo == 0, hi == _EB)

        @pl.when(full)
        def _():
          def grp_body(g, accs):
            fv = fnb[b][pl.ds(g * 16, 16)]
            el0 = g * 16
            for l in range(16):
              accs = edge_step(b, el0 + l, fv[l], accs)
            return accs

          store_accs(lax.fori_loop(0, _EB // 16, grp_body, load_accs()))

        @pl.when(jnp.logical_not(full))
        def _():
          def e_body(el, accs):
            v = fnb[b][pl.ds(el, 16)][0]
            return edge_step(b, el, v, accs)

          store_accs(lax.fori_loop(lo, hi, e_body, load_accs()))

      def quad_body(t, carry):
        for b in range(4):
          kb = 4 * t + b
          b3 = (b + 3) % 4
          wait_gidx(b3)
          issue_dat(b3, base + (kb + 3) * _EB)
          issue_fnb(b3, base + (kb + 3) * _EB)
          wait_dat(b)
          wait_fnb(b)
          issue_gidx(b, base + (kb + 4) * _EB)
          process(kb, b)
        return carry

      lax.fori_loop(0, nquad, quad_body, 0)

      for b in range(3):
        wait_dat(b)
        wait_fnb(b)
      wait_gidx(3)

      pltpu.sync_copy(sumb, s_out.at[pl.ds(nlo, _SUB)])
      pltpu.sync_copy(sqb, q_out.at[pl.ds(nlo, _SUB)])
      pltpu.sync_copy(mxb, mx_out.at[pl.ds(nlo, _SUB)])
      pltpu.sync_copy(mnb, mn_out.at[pl.ds(nlo, _SUB)])
      if want_cnt:
        pltpu.sync_copy(cntb, cnt_out.at[pl.ds(nlo, _SUB)])
      return 0

    lax.fori_loop(0, _NSB, sub_body, 0)

  return pl.kernel(body, out_type=tuple(out_type), mesh=mesh,
                   scratch_types=scratch)


def _proj_g_body(h_ref, w1_ref, w2_ref, g1_ref, g2_ref):
  h = h_ref[...]
  g1_ref[...] = jnp.dot(h, w1_ref[...], preferred_element_type=jnp.float32)
  g2_ref[...] = jnp.dot(h, w2_ref[...], preferred_element_type=jnp.float32)


def _proj_g(h, w1, w2):
  grid = (_NPAD // _NBLK,)
  return pl.pallas_call(
      _proj_g_body,
      grid=grid,
      in_specs=[
          pl.BlockSpec((_NBLK, _D), lambda i: (i, 0)),
          pl.BlockSpec((_D, _D), lambda i: (0, 0)),
          pl.BlockSpec((_D, _D), lambda i: (0, 0)),
      ],
      out_specs=[
          pl.BlockSpec((_NBLK, _D), lambda i: (i, 0)),
          pl.BlockSpec((_NBLK, _D), lambda i: (i, 0)),
      ],
      out_shape=[
          jax.ShapeDtypeStruct((_NPAD, _D), jnp.float32),
          jax.ShapeDtypeStruct((_NPAD, _D), jnp.float32),
      ],
  )(h, w1, w2)


def _proj_q_body(e_ref, w3_ref, b_ref, q_ref):
  q_ref[...] = (
      jnp.dot(e_ref[...], w3_ref[...], preferred_element_type=jnp.float32)
      + b_ref[...])


def _proj_q(e, w3, b):
  grid = (_EPAD // _EBLK,)
  return pl.pallas_call(
      _proj_q_body,
      grid=grid,
      in_specs=[
          pl.BlockSpec((_EBLK, _D), lambda i: (i, 0)),
          pl.BlockSpec((_D, _D), lambda i: (0, 0)),
          pl.BlockSpec((1, _D), lambda i: (0, 0)),
      ],
      out_specs=pl.BlockSpec((_EBLK, _D), lambda i: (i, 0)),
      out_shape=jax.ShapeDtypeStruct((_EPAD, _D), jnp.float32),
  )(e, w3, b)


def _post_body(h_ref, s_ref, q_ref, mx_ref, mn_ref, cnt_ref, sn_ref,
               wp_ref, bp_ref, post_ref, cs_ref, cq_ref):
  i = pl.program_id(0)
  cnt = cnt_ref[:, 0:1]
  pos = cnt > 0.0
  cnt_c = jnp.maximum(cnt, 1.0)
  inv = 1.0 / cnt_c
  mean = s_ref[...] * inv
  sq = q_ref[...] * inv
  std = jnp.sqrt(jnp.maximum(sq - mean * mean, 0.0) + 1e-5)
  mx = jnp.where(pos, mx_ref[...], 0.0)
  mn = jnp.where(pos, mn_ref[...], 0.0)
  logd = jnp.log(cnt + 1.0)
  amp = logd * (1.0 / _AVG_D_LOG)
  att = jnp.where(pos, _AVG_D_LOG / jnp.maximum(logd, 1e-6), 0.0)

  blocks = [h_ref[...], mean, mx, mn, std,
            mean * amp, mx * amp, mn * amp, std * amp,
            mean * att, mx * att, mn * att, std * att]
  acc = jnp.broadcast_to(bp_ref[...], (_NBLK, _D))
  for k in range(13):
    acc = acc + jnp.dot(blocks[k], wp_ref[k],
                        preferred_element_type=jnp.float32)
  post = acc * sn_ref[...]
  post_ref[...] = post
  cs = jnp.sum(post, axis=0, keepdims=True)
  cq = jnp.sum(post * post, axis=0, keepdims=True)

  @pl.when(i == 0)
  def _():
    cs_ref[...] = cs
    cq_ref[...] = cq

  @pl.when(i > 0)
  def _():
    cs_ref[...] = cs_ref[...] + cs
    cq_ref[...] = cq_ref[...] + cq


def _post(h, s, q, mx, mn, cnt, sn, wp, bp):
  grid = (_NPAD // _NBLK,)
  nspec = pl.BlockSpec((_NBLK, _D), lambda i: (i, 0))
  return pl.pallas_call(
      _post_body,
      grid=grid,
      in_specs=[
          nspec, nspec, nspec, nspec, nspec,
          pl.BlockSpec((_NBLK, 16), lambda i: (i, 0)),
          pl.BlockSpec((_NBLK, 1), lambda i: (i, 0)),
          pl.BlockSpec((13, _D, _D), lambda i: (0, 0, 0)),
          pl.BlockSpec((1, _D), lambda i: (0, 0)),
      ],
      out_specs=[
          nspec,
          pl.BlockSpec((1, _D), lambda i: (0, 0)),
          pl.BlockSpec((1, _D), lambda i: (0, 0)),
      ],
      out_shape=[
          jax.ShapeDtypeStruct((_NPAD, _D), jnp.float32),
          jax.ShapeDtypeStruct((1, _D), jnp.float32),
          jax.ShapeDtypeStruct((1, _D), jnp.float32),
      ],
  )(h, s, q, mx, mn, cnt, sn, wp, bp)


def _bnres_body(h_ref, post_ref, cs_ref, cq_ref, g_ref, b_ref, out_ref):
  mu = cs_ref[...] * (1.0 / _N)
  var = cq_ref[...] * (1.0 / _N) - mu * mu
  scale = g_ref[...] / jnp.sqrt(var + 1e-5)
  out_ref[...] = h_ref[...] + (post_ref[...] - mu) * scale + b_ref[...]


def _bnres(h, post, cs, cq, g, b):
  grid = (_NPAD // _NBLK,)
  nspec = pl.BlockSpec((_NBLK, _D), lambda i: (i, 0))
  wspec = pl.BlockSpec((1, _D), lambda i: (0, 0))
  return pl.pallas_call(
      _bnres_body,
      grid=grid,
      in_specs=[nspec, nspec, wspec, wspec, wspec, wspec],
      out_specs=nspec,
      out_shape=jax.ShapeDtypeStruct((_NPAD, _D), jnp.float32),
  )(h, post, cs, cq, g, b)


def _readout_body(h_ref, w0_ref, b0_ref, w1_ref, b1_ref, w2_ref, b2_ref,
                  out_ref):
  x = jnp.dot(h_ref[...], w0_ref[...], preferred_element_type=jnp.float32)
  x = jnp.maximum(x + b0_ref[...], 0.0)
  x = jnp.dot(x, w1_ref[...], preferred_element_type=jnp.float32)
  x = jnp.maximum(x + b1_ref[...], 0.0)
  x = jnp.dot(x, w2_ref[...], preferred_element_type=jnp.float32)
  out_ref[...] = x + b2_ref[...]


def _readout(h, w0, b0, w1, b1, w2, b2):
  grid = (_NPAD // _NBLK,)
  return pl.pallas_call(
      _readout_body,
      grid=grid,
      in_specs=[
          pl.BlockSpec((_NBLK, _D), lambda i: (i, 0)),
          pl.BlockSpec((_D, 64), lambda i: (0, 0)),
          pl.BlockSpec((1, 64), lambda i: (0, 0)),
          pl.BlockSpec((64, 32), lambda i: (0, 0)),
          pl.BlockSpec((1, 32), lambda i: (0, 0)),
          pl.BlockSpec((32, 16), lambda i: (0, 0)),
          pl.BlockSpec((1, 16), lambda i: (0, 0)),
      ],
      out_specs=pl.BlockSpec((_NBLK, 16), lambda i: (i, 0)),
      out_shape=jax.ShapeDtypeStruct((_NPAD, 16), jnp.float32),
  )(h, w0, b0, w1, b1, w2, b2)


def kernel(h, e, snorm_n, snorm_e, edge_index, W_pre, b_pre, W_post, b_post,
           gamma, beta, Wr0, br0, Wr1, br1, Wr2, br2):
  src = edge_index[0].astype(jnp.int32)
  dst = edge_index[1].astype(jnp.int32)

  sd, ss, pm = lax.sort((dst, src, jnp.arange(_E, dtype=jnp.int32)),
                        num_keys=1)
  breaks = jnp.minimum(jnp.arange(0, _NPAD + 1, _SUB, dtype=jnp.int32), _N)
  rp = jnp.searchsorted(sd, breaks).astype(jnp.int32)
  tr = jnp.zeros((_NT, 32), jnp.int32)
  for k in range(_NSB + 1):
    tr = tr.at[:, k].set(rp[k:k + _NSB * (_NT - 1) + 1:_NSB])

  bnd = jnp.concatenate(
      [(sd[1:] != sd[:-1]), jnp.ones((1,), jnp.bool_)]).astype(jnp.int32)
  fnv = sd * 2 + bnd

  zpad = jnp.zeros((_EPAD - _E,), jnp.int32)
  ss_p = jnp.concatenate([ss, zpad])
  fn_p = jnp.concatenate([fnv, zpad])

  hp = jnp.concatenate([h, jnp.zeros((_NPAD - _N, _D), jnp.float32)])
  snp = jnp.concatenate([snorm_n, jnp.zeros((_NPAD - _N, 1), jnp.float32)])

  e_wide = jnp.concatenate(
      [e, jnp.zeros((_E, _D - _EDIM), jnp.float32)], axis=1)
  es = _make_sc_permute()(e_wide, pm)
  sc_edge0 = _make_sc_edge(True)
  sc_edge = _make_sc_edge(False)

  cnt = None
  for i in range(_L):
    w1 = W_pre[i, :_D]
    w2 = W_pre[i, _D:2 * _D]
    w3 = jnp.concatenate(
        [W_pre[i, 2 * _D:], jnp.zeros((_D - _EDIM, _D), jnp.float32)])
    bpre = b_pre[i].reshape(1, _D)
    g1, g2 = _proj_g(hp, w1, w2)
    q = _proj_q(es, w3, bpre)
    if i == 0:
      s, sq, mx, mn, cnt = sc_edge0(g1, g2, q, ss_p, fn_p, tr)
    else:
      s, sq, mx, mn = sc_edge(g1, g2, q, ss_p, fn_p, tr)
    wp = W_post[i].reshape(13, _D, _D)
    bp = b_post[i].reshape(1, _D)
    post, cs, cq = _post(hp, s, sq, mx, mn, cnt, snp, wp, bp)
    hp = _bnres(hp, post, cs, cq, gamma[i].reshape(1, _D),
                beta[i].reshape(1, _D))

  w2r = jnp.concatenate([Wr2, jnp.zeros((32, 6), jnp.float32)], axis=1)
  b2r = jnp.concatenate([br2, jnp.zeros((6,), jnp.float32)]).reshape(1, 16)
  out = _readout(hp, Wr0, br0.reshape(1, 64), Wr1, br1.reshape(1, 32),
                 w2r, b2r)
  return out[:_N, :10]


# speedup vs baseline: 1.6527x; 1.6527x over previous
import functools

import jax
import jax.numpy as jnp
from jax import lax
from jax.experimental import pallas as pl
from jax.experimental.pallas import tpu as pltpu
from jax.experimental.pallas import tpu_sc as plsc

_N = 10000
_E = 160000
_D = 128
_EDIM = 16
_L = 4
_AVG_D_LOG = 2.772588722239781

_NT = 32
_NPT = 360
_NPAD = _NT * _NPT
_NSB = 3
_SUB = 120
_EB = 64
_EPAD = 162000
_EPB = 200

_NBLK = 1440
_EBLK = 2000


def _make_sc_permute():
  mesh = plsc.VectorSubcoreMesh(
      core_axis_name="c", subcore_axis_name="s", num_cores=2,
      num_subcores=16)
  ept = _E // _NT

  def body(e, pm, es, pmb, erb, sem):
    wid = lax.axis_index("s") * 2 + lax.axis_index("c")
    r0 = wid * ept

    def blk(k, carry):
      bs = r0 + k * _EPB
      pltpu.sync_copy(pm.at[pl.ds(bs, _EPB)], pmb)
      pltpu.async_copy(e.at[pmb], erb, sem).wait()
      pltpu.sync_copy(erb, es.at[pl.ds(bs, _EPB)])
      return carry

    lax.fori_loop(0, ept // _EPB, blk, 0)

  return pl.kernel(
      body,
      out_type=jax.ShapeDtypeStruct((_EPAD, _D), jnp.float32),
      mesh=mesh,
      scratch_types=[
          pltpu.VMEM((_EPB,), jnp.int32),
          pltpu.VMEM((_EPB, _D), jnp.float32),
          pltpu.SemaphoreType.DMA,
      ])


def _make_sc_edge(want_cnt):
  mesh = plsc.VectorSubcoreMesh(
      core_axis_name="c", subcore_axis_name="s", num_cores=2,
      num_subcores=16)
  out_type = [jax.ShapeDtypeStruct((_NPAD, _D), jnp.float32) for _ in range(4)]
  if want_cnt:
    out_type.append(jax.ShapeDtypeStruct((_NPAD, 16), jnp.float32))
  scratch = [
      [pltpu.VMEM((_EB,), jnp.int32)] * 2,
      [pltpu.VMEM((_EB + 16,), jnp.int32)] * 2,
      [pltpu.VMEM((_EB, _D), jnp.float32)] * 2,
      [pltpu.VMEM((_EB, _D), jnp.float32)] * 2,
      pltpu.VMEM((_SUB, _D), jnp.float32),
      pltpu.VMEM((_SUB, _D), jnp.float32),
      pltpu.VMEM((_SUB, _D), jnp.float32),
      pltpu.VMEM((_SUB, _D), jnp.float32),
      pltpu.VMEM((_SUB, _D), jnp.float32),
      pltpu.VMEM((_SUB, 16), jnp.float32),
      pltpu.VMEM((40, 16), jnp.float32),
      pltpu.VMEM((1, 32), jnp.int32),
      [pltpu.SemaphoreType.DMA] * 2,
      [pltpu.SemaphoreType.DMA] * 2,
      [pltpu.SemaphoreType.DMA] * 2,
      pltpu.SemaphoreType.DMA,
  ]

  def body(g1, g2, q, ss, fn, tr, *refs):
    nout = 5 if want_cnt else 4
    outs = refs[:nout]
    (ssb, fnb, g1b, qb, g2s, sumb, sqb, mxb, mnb, cntb, accbuf, rngb,
     sem_gidx, sem_fnb, sem_dat, sem_g2s) = refs[nout:]
    if want_cnt:
      s_out, q_out, mx_out, mn_out, cnt_out = outs
    else:
      s_out, q_out, mx_out, mn_out = outs

    wid = lax.axis_index("s") * 2 + lax.axis_index("c")
    pltpu.sync_copy(tr.at[wid], rngb.at[0])

    zeros16 = jnp.zeros((16,), jnp.float32)
    big16 = jnp.full((16,), 1e30, jnp.float32)

    def issue_gidx(b, bs):
      pltpu.async_copy(ss.at[pl.ds(bs, _EB)], ssb[b], sem_gidx[b])

    def wait_gidx(b):
      pltpu.make_async_copy(ss.at[pl.ds(0, _EB)], ssb[b], sem_gidx[b]).wait()

    def issue_fnb(b, bs):
      pltpu.async_copy(fn.at[pl.ds(bs, _EB)], fnb[b].at[pl.ds(0, _EB)],
                       sem_fnb[b])

    def wait_fnb(b):
      pltpu.make_async_copy(fn.at[pl.ds(0, _EB)], fnb[b].at[pl.ds(0, _EB)],
                            sem_fnb[b]).wait()

    def sub_body(j, _):
      nlo = wid * _NPT + j * _SUB

      cpg = pltpu.async_copy(g2.at[pl.ds(nlo, _SUB)], g2s, sem_g2s)

      def init_body(r, carry):
        for c in range(8):
          sl = pl.ds(c * 16, 16)
          sumb[r, sl] = zeros16
          sqb[r, sl] = zeros16
          mxb[r, sl] = zeros16
          mnb[r, sl] = big16
        if want_cnt:
          cntb[r, pl.ds(0, 16)] = zeros16
        return carry

      lax.fori_loop(0, _SUB, init_body, 0)

      rv = rngb[0, pl.ds(j, 16)]
      e0 = rv[0]
      e1 = rv[1]
      base = (e0 // _EB) * _EB

      def issue_dat(b, bs, kb):
        @pl.when(base + kb * _EB < e1)
        def _():
          pltpu.async_copy(g1.at[ssb[b]], g1b[b], sem_dat[b])
          pltpu.async_copy(q.at[pl.ds(bs, _EB)], qb[b], sem_dat[b])

      def wait_dat(b, kb):
        @pl.when(base + kb * _EB < e1)
        def _():
          pltpu.make_async_copy(g1.at[ssb[b]], g1b[b], sem_dat[b]).wait()
          pltpu.make_async_copy(q.at[pl.ds(0, _EB)], qb[b],
                                sem_dat[b]).wait()
      nblk = (e1 - base + (_EB - 1)) // _EB
      npair = jnp.maximum((nblk + 1) // 2, 1)

      issue_gidx(0, base)
      wait_gidx(0)
      issue_dat(0, base, 0)
      issue_fnb(0, base)
      issue_gidx(1, base + _EB)
      cpg.wait()

      sl16 = pl.ds(0, 16)

      def store_accs(accs):
        sacc, qacc, xacc, nacc, cacc = accs
        for c in range(8):
          accbuf[c, sl16] = sacc[c]
          accbuf[8 + c, sl16] = qacc[c]
          accbuf[16 + c, sl16] = xacc[c]
          accbuf[24 + c, sl16] = nacc[c]
        accbuf[32, sl16] = cacc

      def load_accs():
        return ([accbuf[c, sl16] for c in range(8)],
                [accbuf[8 + c, sl16] for c in range(8)],
                [accbuf[16 + c, sl16] for c in range(8)],
                [accbuf[24 + c, sl16] for c in range(8)],
                accbuf[32, sl16])

      store_accs(([zeros16] * 8, [zeros16] * 8, [zeros16] * 8, [big16] * 8,
                  zeros16))

      def edge_step(b, el, v, accs):
        sacc, qacc, xacc, nacc, cacc = accs
        row = (v >> 1) - nlo
        is_b = (v & 1) == 1
        sacc2, qacc2, xacc2, nacc2 = [], [], [], []
        for c in range(8):
          sl = pl.ds(c * 16, 16)
          m = g1b[b][el, sl] + g2s[row, sl] + qb[b][el, sl]
          m = jnp.maximum(m, 0.0)
          sacc2.append(sacc[c] + m)
          qacc2.append(qacc[c] + m * m)
          xacc2.append(jnp.maximum(xacc[c], m))
          nacc2.append(jnp.minimum(nacc[c], m))
        cacc2 = cacc + jnp.maximum(m * 0.0, 1.0)

        @pl.when(is_b)
        def _():
          for c in range(8):
            sl = pl.ds(c * 16, 16)
            sumb[row, sl] = sacc2[c]
            sqb[row, sl] = qacc2[c]
            mxb[row, sl] = xacc2[c]
            mnb[row, sl] = nacc2[c]
          if want_cnt:
            cntb[row, pl.ds(0, 16)] = cacc2

        sacc3 = [jnp.where(is_b, zeros16, x) for x in sacc2]
        qacc3 = [jnp.where(is_b, zeros16, x) for x in qacc2]
        xacc3 = [jnp.where(is_b, zeros16, x) for x in xacc2]
        nacc3 = [jnp.where(is_b, big16, x) for x in nacc2]
        cacc3 = jnp.where(is_b, zeros16, cacc2)
        return (sacc3, qacc3, xacc3, nacc3, cacc3)

      def process(kb, b):
        bs = base + kb * _EB
        lo = jnp.maximum(e0 - bs, 0)
        hi = jnp.minimum(e1 - bs, _EB)
        full = jnp.logical_and(lo == 0, hi == _EB)

        @pl.when(full)
        def _():
          def grp_body(g, accs):
            fv = fnb[b][pl.ds(g * 16, 16)]
            el0 = g * 16
            for l in range(16):
              accs = edge_step(b, el0 + l, fv[l], accs)
            return accs

          store_accs(lax.fori_loop(0, _EB // 16, grp_body, load_accs()))

        @pl.when(jnp.logical_not(full))
        def _():
          def e_body(el, accs):
            v = fnb[b][pl.ds(el, 16)][0]
            return edge_step(b, el, v, accs)

          store_accs(lax.fori_loop(lo, hi, e_body, load_accs()))

      def pair_body(t, carry):
        for b in range(2):
          kb = 2 * t + b
          o = 1 - b
          wait_gidx(o)
          issue_dat(o, base + (kb + 1) * _EB, kb + 1)
          issue_fnb(o, base + (kb + 1) * _EB)
          wait_dat(b, kb)
          wait_fnb(b)
          issue_gidx(b, base + (kb + 2) * _EB)
          process(kb, b)
        return carry

      lax.fori_loop(0, npair, pair_body, 0)

      wait_dat(0, 2 * npair)
      wait_fnb(0)
      wait_gidx(1)

      pltpu.sync_copy(sumb, s_out.at[pl.ds(nlo, _SUB)])
      pltpu.sync_copy(sqb, q_out.at[pl.ds(nlo, _SUB)])
      pltpu.sync_copy(mxb, mx_out.at[pl.ds(nlo, _SUB)])
      pltpu.sync_copy(mnb, mn_out.at[pl.ds(nlo, _SUB)])
      if want_cnt:
        pltpu.sync_copy(cntb, cnt_out.at[pl.ds(nlo, _SUB)])
      return 0

    lax.fori_loop(0, _NSB, sub_body, 0)

  return pl.kernel(body, out_type=tuple(out_type), mesh=mesh,
                   scratch_types=scratch)


def _proj_g_body(h_ref, w1_ref, w2_ref, g1_ref, g2_ref):
  h = h_ref[...]
  g1_ref[...] = jnp.dot(h, w1_ref[...], preferred_element_type=jnp.float32)
  g2_ref[...] = jnp.dot(h, w2_ref[...], preferred_element_type=jnp.float32)


def _proj_g(h, w1, w2):
  grid = (_NPAD // _NBLK,)
  return pl.pallas_call(
      _proj_g_body,
      grid=grid,
      in_specs=[
          pl.BlockSpec((_NBLK, _D), lambda i: (i, 0)),
          pl.BlockSpec((_D, _D), lambda i: (0, 0)),
          pl.BlockSpec((_D, _D), lambda i: (0, 0)),
      ],
      out_specs=[
          pl.BlockSpec((_NBLK, _D), lambda i: (i, 0)),
          pl.BlockSpec((_NBLK, _D), lambda i: (i, 0)),
      ],
      out_shape=[
          jax.ShapeDtypeStruct((_NPAD, _D), jnp.float32),
          jax.ShapeDtypeStruct((_NPAD, _D), jnp.float32),
      ],
  )(h, w1, w2)


def _proj_q_body(e_ref, w3_ref, b_ref, q_ref):
  q_ref[...] = (
      jnp.dot(e_ref[...], w3_ref[...], preferred_element_type=jnp.float32)
      + b_ref[...])


def _proj_q(e, w3, b):
  grid = (_EPAD // _EBLK,)
  return pl.pallas_call(
      _proj_q_body,
      grid=grid,
      in_specs=[
          pl.BlockSpec((_EBLK, _D), lambda i: (i, 0)),
          pl.BlockSpec((_D, _D), lambda i: (0, 0)),
          pl.BlockSpec((1, _D), lambda i: (0, 0)),
      ],
      out_specs=pl.BlockSpec((_EBLK, _D), lambda i: (i, 0)),
      out_shape=jax.ShapeDtypeStruct((_EPAD, _D), jnp.float32),
  )(e, w3, b)


def _post_body(h_ref, s_ref, q_ref, mx_ref, mn_ref, cnt_ref, sn_ref,
               wp_ref, bp_ref, post_ref, cs_ref, cq_ref):
  i = pl.program_id(0)
  cnt = cnt_ref[:, 0:1]
  pos = cnt > 0.0
  cnt_c = jnp.maximum(cnt, 1.0)
  inv = 1.0 / cnt_c
  mean = s_ref[...] * inv
  sq = q_ref[...] * inv
  std = jnp.sqrt(jnp.maximum(sq - mean * mean, 0.0) + 1e-5)
  mx = jnp.where(pos, mx_ref[...], 0.0)
  mn = jnp.where(pos, mn_ref[...], 0.0)
  logd = jnp.log(cnt + 1.0)
  amp = logd * (1.0 / _AVG_D_LOG)
  att = jnp.where(pos, _AVG_D_LOG / jnp.maximum(logd, 1e-6), 0.0)

  blocks = [h_ref[...], mean, mx, mn, std,
            mean * amp, mx * amp, mn * amp, std * amp,
            mean * att, mx * att, mn * att, std * att]
  acc = jnp.broadcast_to(bp_ref[...], (_NBLK, _D))
  for k in range(13):
    acc = acc + jnp.dot(blocks[k], wp_ref[k],
                        preferred_element_type=jnp.float32)
  post = acc * sn_ref[...]
  post_ref[...] = post
  cs = jnp.sum(post, axis=0, keepdims=True)
  cq = jnp.sum(post * post, axis=0, keepdims=True)

  @pl.when(i == 0)
  def _():
    cs_ref[...] = cs
    cq_ref[...] = cq

  @pl.when(i > 0)
  def _():
    cs_ref[...] = cs_ref[...] + cs
    cq_ref[...] = cq_ref[...] + cq


def _post(h, s, q, mx, mn, cnt, sn, wp, bp):
  grid = (_NPAD // _NBLK,)
  nspec = pl.BlockSpec((_NBLK, _D), lambda i: (i, 0))
  return pl.pallas_call(
      _post_body,
      grid=grid,
      in_specs=[
          nspec, nspec, nspec, nspec, nspec,
          pl.BlockSpec((_NBLK, 16), lambda i: (i, 0)),
          pl.BlockSpec((_NBLK, 1), lambda i: (i, 0)),
          pl.BlockSpec((13, _D, _D), lambda i: (0, 0, 0)),
          pl.BlockSpec((1, _D), lambda i: (0, 0)),
      ],
      out_specs=[
          nspec,
          pl.BlockSpec((1, _D), lambda i: (0, 0)),
          pl.BlockSpec((1, _D), lambda i: (0, 0)),
      ],
      out_shape=[
          jax.ShapeDtypeStruct((_NPAD, _D), jnp.float32),
          jax.ShapeDtypeStruct((1, _D), jnp.float32),
          jax.ShapeDtypeStruct((1, _D), jnp.float32),
      ],
  )(h, s, q, mx, mn, cnt, sn, wp, bp)


def _bnres_body(h_ref, post_ref, cs_ref, cq_ref, g_ref, b_ref, out_ref):
  mu = cs_ref[...] * (1.0 / _N)
  var = cq_ref[...] * (1.0 / _N) - mu * mu
  scale = g_ref[...] / jnp.sqrt(var + 1e-5)
  out_ref[...] = h_ref[...] + (post_ref[...] - mu) * scale + b_ref[...]


def _bnres(h, post, cs, cq, g, b):
  grid = (_NPAD // _NBLK,)
  nspec = pl.BlockSpec((_NBLK, _D), lambda i: (i, 0))
  wspec = pl.BlockSpec((1, _D), lambda i: (0, 0))
  return pl.pallas_call(
      _bnres_body,
      grid=grid,
      in_specs=[nspec, nspec, wspec, wspec, wspec, wspec],
      out_specs=nspec,
      out_shape=jax.ShapeDtypeStruct((_NPAD, _D), jnp.float32),
  )(h, post, cs, cq, g, b)


def _readout_body(h_ref, w0_ref, b0_ref, w1_ref, b1_ref, w2_ref, b2_ref,
                  out_ref):
  x = jnp.dot(h_ref[...], w0_ref[...], preferred_element_type=jnp.float32)
  x = jnp.maximum(x + b0_ref[...], 0.0)
  x = jnp.dot(x, w1_ref[...], preferred_element_type=jnp.float32)
  x = jnp.maximum(x + b1_ref[...], 0.0)
  x = jnp.dot(x, w2_ref[...], preferred_element_type=jnp.float32)
  out_ref[...] = x + b2_ref[...]


def _readout(h, w0, b0, w1, b1, w2, b2):
  grid = (_NPAD // _NBLK,)
  return pl.pallas_call(
      _readout_body,
      grid=grid,
      in_specs=[
          pl.BlockSpec((_NBLK, _D), lambda i: (i, 0)),
          pl.BlockSpec((_D, 64), lambda i: (0, 0)),
          pl.BlockSpec((1, 64), lambda i: (0, 0)),
          pl.BlockSpec((64, 32), lambda i: (0, 0)),
          pl.BlockSpec((1, 32), lambda i: (0, 0)),
          pl.BlockSpec((32, 16), lambda i: (0, 0)),
          pl.BlockSpec((1, 16), lambda i: (0, 0)),
      ],
      out_specs=pl.BlockSpec((_NBLK, 16), lambda i: (i, 0)),
      out_shape=jax.ShapeDtypeStruct((_NPAD, 16), jnp.float32),
  )(h, w0, b0, w1, b1, w2, b2)


def kernel(h, e, snorm_n, snorm_e, edge_index, W_pre, b_pre, W_post, b_post,
           gamma, beta, Wr0, br0, Wr1, br1, Wr2, br2):
  src = edge_index[0].astype(jnp.int32)
  dst = edge_index[1].astype(jnp.int32)

  sd, ss, pm = lax.sort((dst, src, jnp.arange(_E, dtype=jnp.int32)),
                        num_keys=1)
  breaks = jnp.minimum(jnp.arange(0, _NPAD + 1, _SUB, dtype=jnp.int32), _N)
  rp = jnp.searchsorted(sd, breaks).astype(jnp.int32)
  tr = jnp.zeros((_NT, 32), jnp.int32)
  for k in range(_NSB + 1):
    tr = tr.at[:, k].set(rp[k:k + _NSB * (_NT - 1) + 1:_NSB])

  bnd = jnp.concatenate(
      [(sd[1:] != sd[:-1]), jnp.ones((1,), jnp.bool_)]).astype(jnp.int32)
  fnv = sd * 2 + bnd

  zpad = jnp.zeros((_EPAD - _E,), jnp.int32)
  ss_p = jnp.concatenate([ss, zpad])
  fn_p = jnp.concatenate([fnv, zpad])

  hp = jnp.concatenate([h, jnp.zeros((_NPAD - _N, _D), jnp.float32)])
  snp = jnp.concatenate([snorm_n, jnp.zeros((_NPAD - _N, 1), jnp.float32)])

  e_wide = jnp.concatenate(
      [e, jnp.zeros((_E, _D - _EDIM), jnp.float32)], axis=1)
  es = _make_sc_permute()(e_wide, pm)
  sc_edge0 = _make_sc_edge(True)
  sc_edge = _make_sc_edge(False)

  cnt = None
  for i in range(_L):
    w1 = W_pre[i, :_D]
    w2 = W_pre[i, _D:2 * _D]
    w3 = jnp.concatenate(
        [W_pre[i, 2 * _D:], jnp.zeros((_D - _EDIM, _D), jnp.float32)])
    bpre = b_pre[i].reshape(1, _D)
    g1, g2 = _proj_g(hp, w1, w2)
    q = _proj_q(es, w3, bpre)
    if i == 0:
      s, sq, mx, mn, cnt = sc_edge0(g1, g2, q, ss_p, fn_p, tr)
    else:
      s, sq, mx, mn = sc_edge(g1, g2, q, ss_p, fn_p, tr)
    wp = W_post[i].reshape(13, _D, _D)
    bp = b_post[i].reshape(1, _D)
    post, cs, cq = _post(hp, s, sq, mx, mn, cnt, snp, wp, bp)
    hp = _bnres(hp, post, cs, cq, gamma[i].reshape(1, _D),
                beta[i].reshape(1, _D))

  w2r = jnp.concatenate([Wr2, jnp.zeros((32, 6), jnp.float32)], axis=1)
  b2r = jnp.concatenate([br2, jnp.zeros((6,), jnp.float32)]).reshape(1, 16)
  out = _readout(hp, Wr0, br0.reshape(1, 64), Wr1, br1.reshape(1, 32),
                 w2r, b2r)
  return out[:_N, :10]
